# baseline (speedup 1.0000x reference)
import jax
import jax.numpy as jnp
from jax.experimental import pallas as pl
from jax.experimental.pallas import tpu as pltpu


def _slab_kernel(x_ref, out_ref):
    jj = jax.lax.broadcasted_iota(jnp.int32, (128, 128), 0)
    kk = jax.lax.broadcasted_iota(jnp.int32, (128, 128), 1)
    interior = (jj < 127) & (kk < 127)

    def body(i, carry):
        acc, a = carry
        xn = x_ref[0, i + 1]
        dx = xn - a
        ay = jnp.concatenate([a[1:, :], a[:1, :]], axis=0)
        az = jnp.concatenate([a[:, 1:], a[:, :1]], axis=1)
        dy = ay - a
        dz = az - a
        s = dx * dx + dy * dy + dz * dz
        acc = acc + jnp.where(interior, s, 0.0)
        return acc, xn

    acc0 = jnp.zeros((128, 128), dtype=jnp.float32)
    a0 = x_ref[0, 0]
    acc, _ = jax.lax.fori_loop(0, 127, body, (acc0, a0))
    r = jnp.sum(acc, axis=0, keepdims=True)
    out_ref[0] = jnp.sum(r, axis=1, keepdims=True)


def kernel(mesh):
    slabs = mesh.reshape(12, 128, 128, 128)
    partials = pl.pallas_call(
        _slab_kernel,
        grid=(12,),
        in_specs=[
            pl.BlockSpec((1, 128, 128, 128), lambda i: (i, 0, 0, 0)),
        ],
        out_specs=pl.BlockSpec((1, 1, 1), lambda i: (i, 0, 0)),
        out_shape=jax.ShapeDtypeStruct((12, 1, 1), jnp.float32),
        compiler_params=pltpu.CompilerParams(
            dimension_semantics=("parallel",),
        ),
    )(slabs)
    return jnp.sum(partials) / jnp.float32(12.0)


# speedup vs baseline: 2.9752x; 2.9752x over previous
import jax
import jax.numpy as jnp
from jax.experimental import pallas as pl
from jax.experimental.pallas import tpu as pltpu


def _slab_kernel(x_ref, out_ref):
    jj = jax.lax.broadcasted_iota(jnp.int32, (128, 128), 0)
    kk = jax.lax.broadcasted_iota(jnp.int32, (128, 128), 1)
    interior = (jj < 127) & (kk < 127)

    accs = [jnp.zeros((128, 128), dtype=jnp.float32) for _ in range(4)]
    a = x_ref[0, 0]
    for i in range(127):
        xn = x_ref[0, i + 1]
        dx = xn - a
        ay = jnp.concatenate([a[1:, :], a[:1, :]], axis=0)
        az = jnp.concatenate([a[:, 1:], a[:, :1]], axis=1)
        dy = ay - a
        dz = az - a
        s = dx * dx + dy * dy + dz * dz
        accs[i % 4] = accs[i % 4] + jnp.where(interior, s, 0.0)
        a = xn
    acc = (accs[0] + accs[1]) + (accs[2] + accs[3])
    r = jnp.sum(acc, axis=0, keepdims=True)
    out_ref[0] = jnp.sum(r, axis=1, keepdims=True)


def kernel(mesh):
    slabs = mesh.reshape(12, 128, 128, 128)
    partials = pl.pallas_call(
        _slab_kernel,
        grid=(12,),
        in_specs=[
            pl.BlockSpec((1, 128, 128, 128), lambda i: (i, 0, 0, 0)),
        ],
        out_specs=pl.BlockSpec((1, 1, 1), lambda i: (i, 0, 0)),
        out_shape=jax.ShapeDtypeStruct((12, 1, 1), jnp.float32),
        compiler_params=pltpu.CompilerParams(
            dimension_semantics=("parallel",),
        ),
    )(slabs)
    return jnp.sum(partials) / jnp.float32(12.0)


# speedup vs baseline: 3.0592x; 1.0283x over previous
import jax
import jax.numpy as jnp
from jax.experimental import pallas as pl
from jax.experimental.pallas import tpu as pltpu


def _slab_kernel(x_ref, out_ref):
    jj = jax.lax.broadcasted_iota(jnp.int32, (128, 128), 0)
    kk = jax.lax.broadcasted_iota(jnp.int32, (128, 128), 1)
    interior = (jj < 127) & (kk < 127)

    accs = [jnp.zeros((128, 128), dtype=jnp.float32) for _ in range(2)]
    a = x_ref[0, 0]
    for i in range(127):
        xn = x_ref[0, i + 1]
        dx = xn - a
        ay = pltpu.roll(a, 127, axis=0)
        az = pltpu.roll(a, 127, axis=1)
        dy = ay - a
        dz = az - a
        s = dx * dx + dy * dy + dz * dz
        accs[i % 2] = accs[i % 2] + s
        a = xn
    acc = accs[0] + accs[1]
    acc = jnp.where(interior, acc, 0.0)
    r = jnp.sum(acc, axis=0, keepdims=True)
    out_ref[0] = jnp.sum(r, axis=1, keepdims=True)


def kernel(mesh):
    slabs = mesh.reshape(12, 128, 128, 128)
    partials = pl.pallas_call(
        _slab_kernel,
        grid=(12,),
        in_specs=[
            pl.BlockSpec((1, 128, 128, 128), lambda i: (i, 0, 0, 0)),
        ],
        out_specs=pl.BlockSpec((1, 1, 1), lambda i: (i, 0, 0)),
        out_shape=jax.ShapeDtypeStruct((12, 1, 1), jnp.float32),
        compiler_params=pltpu.CompilerParams(
            dimension_semantics=("parallel",),
        ),
    )(slabs)
    return jnp.sum(partials) / jnp.float32(12.0)
